# baseline (speedup 1.0000x reference)
import functools

import jax
import jax.numpy as jnp
from jax import lax
from jax.experimental import pallas as pl
from jax.experimental.pallas import tpu as pltpu
from jax.experimental.pallas import tpu_sc as plsc

BATCH = 4096
SEQ = 200
DIM = 64

_info = plsc.get_sparse_core_info()
NC, NS, NL = _info.num_cores, _info.num_subcores, _info.num_lanes
NW = NC * NS
NS1 = SEQ // 8
HSROWS = 4 * 128
NCH = 2 * NS1


def _sc_embed(idx_hbm, tok_hbm, pos_hbm, out_hbm, idx_bufs, pos_v,
              row_bufs, t_bufs, idx_sems, in_sems, t_sems):
    wid = lax.axis_index("s") * NC + lax.axis_index("c")
    lanes = lax.iota(jnp.int32, NL)
    d1v = [(g * NL + lanes) // 8 for g in range(4)]
    d0v = [(g * NL + lanes) % 8 for g in range(4)]

    def stage_idx(c, buf):
        s1, hs = c // 2, c % 2
        pltpu.async_copy(idx_hbm.at[s1, wid, pl.ds(hs * 512, 512)],
                         idx_bufs[buf], idx_sems[buf])

    def wait_idx(buf):
        pltpu.make_async_copy(idx_hbm.at[0, 0, pl.ds(0, 512)], idx_bufs[buf],
                              idx_sems[buf]).wait()

    def start_gather(buf):
        pltpu.async_copy(tok_hbm.at[idx_bufs[buf]], row_bufs[buf],
                         in_sems[buf])

    def wait_gather(buf):
        pltpu.make_async_copy(tok_hbm.at[idx_bufs[buf]], row_bufs[buf],
                              in_sems[buf]).wait()

    stage_idx(0, 0)
    stage_idx(1, 1)
    pltpu.sync_copy(pos_hbm, pos_v)
    wait_idx(0)
    start_gather(0)

    def octet(s1, par):
        for hs in range(2):
            c = 2 * s1 + hs
            wait_gather(hs)

            @pl.when(c + 1 < NCH)
            def _():
                wait_idx(1 - hs)
                start_gather(1 - hs)

            @pl.when(c + 2 < NCH)
            def _():
                stage_idx(c + 2, hs)

            rows = row_bufs[hs]
            for sl in range(4):
                s = s1 * 8 + hs * 4 + sl
                tb = t_bufs[sl % 2]
                if sl >= 2:
                    pltpu.make_async_copy(tb.at[:, :, pl.ds(0, 128)],
                                          out_hbm.at[0, :, wid],
                                          t_sems[sl % 2]).wait()
                else:
                    @pl.when(c >= 1)
                    def _():
                        pltpu.make_async_copy(tb.at[:, :, pl.ds(0, 128)],
                                              out_hbm.at[0, :, wid],
                                              t_sems[sl % 2]).wait()

                for g in range(4):
                    pv = pos_v[s, pl.ds(g * NL, NL)]
                    dg1, dg0 = d1v[g], d0v[g]

                    @plsc.parallel_loop(0, 128, unroll=8)
                    def _(r):
                        v = rows[sl * 128 + r, pl.ds(g * NL, NL)] + pv
                        plsc.store_scatter(tb, [dg1, dg0, lanes * 0 + r], v)

                pltpu.async_copy(tb.at[:, :, pl.ds(0, 128)],
                                 out_hbm.at[s, :, wid], t_sems[sl % 2])

    def octet_pair(k, _):
        octet(2 * k, 0)
        octet(2 * k + 1, 1)
        return 0

    lax.fori_loop(0, (NS1 - 1) // 2, octet_pair, 0)
    octet(NS1 - 1, 0)

    for i in range(2):
        pltpu.make_async_copy(t_bufs[i].at[:, :, pl.ds(0, 128)],
                              out_hbm.at[0, :, wid], t_sems[i]).wait()


@jax.jit
def kernel(inputs, token_table, position_table):
    idx4 = (inputs.T.astype(jnp.int32)
            .reshape(NS1, 8, NW, 128).transpose(0, 2, 1, 3)
            .reshape(NS1, NW, 8 * 128))
    mesh = plsc.VectorSubcoreMesh(core_axis_name="c", subcore_axis_name="s")
    out5 = pl.kernel(
        _sc_embed,
        mesh=mesh,
        out_type=jax.ShapeDtypeStruct((SEQ, 8, NW, 8, 128), jnp.float32),
        scratch_types=[
            [pltpu.VMEM((512,), jnp.int32) for _ in range(2)],
            pltpu.VMEM((SEQ, DIM), jnp.float32),
            [pltpu.VMEM((HSROWS, DIM), jnp.float32) for _ in range(2)],
            [pltpu.VMEM((8, 8, 129), jnp.float32) for _ in range(2)],
            [pltpu.SemaphoreType.DMA for _ in range(2)],
            [pltpu.SemaphoreType.DMA for _ in range(2)],
            [pltpu.SemaphoreType.DMA for _ in range(2)],
        ],
        compiler_params=pltpu.CompilerParams(use_tc_tiling_on_sc=False,
                                             needs_layout_passes=False),
    )(idx4, token_table, position_table)
    out_phys = out5.transpose(0, 1, 3, 2, 4).reshape(SEQ, DIM, BATCH)
    return out_phys.transpose(2, 0, 1)


# speedup vs baseline: 1.0108x; 1.0108x over previous
import functools

import jax
import jax.numpy as jnp
from jax import lax
from jax.experimental import pallas as pl
from jax.experimental.pallas import tpu as pltpu
from jax.experimental.pallas import tpu_sc as plsc

BATCH = 4096
SEQ = 200
DIM = 64

_info = plsc.get_sparse_core_info()
NC, NS, NL = _info.num_cores, _info.num_subcores, _info.num_lanes
NW = NC * NS
NS1 = SEQ // 8
HSROWS = 4 * 128
NCH = 2 * NS1


def _sc_embed(idx_hbm, tok_hbm, pos_hbm, out_hbm, idx_bufs, pos_v,
              row_bufs, t_bufs, idx_sems, in_sems, t_sems):
    wid = lax.axis_index("s") * NC + lax.axis_index("c")
    lanes = lax.iota(jnp.int32, NL)
    d1v = [(g * NL + lanes) // 8 for g in range(4)]
    d0v = [(g * NL + lanes) % 8 for g in range(4)]

    def stage_idx(c, buf):
        s1, hs = c // 2, c % 2
        pltpu.async_copy(idx_hbm.at[s1, wid, pl.ds(hs * 512, 512)],
                         idx_bufs[buf], idx_sems[buf])

    def wait_idx(buf):
        pltpu.make_async_copy(idx_hbm.at[0, 0, pl.ds(0, 512)], idx_bufs[buf],
                              idx_sems[buf]).wait()

    def start_gather(buf):
        pltpu.async_copy(tok_hbm.at[idx_bufs[buf]], row_bufs[buf],
                         in_sems[buf])

    def wait_gather(buf):
        pltpu.make_async_copy(tok_hbm.at[idx_bufs[buf]], row_bufs[buf],
                              in_sems[buf]).wait()

    stage_idx(0, 0)
    stage_idx(1, 1)
    pltpu.sync_copy(pos_hbm, pos_v)
    wait_idx(0)
    start_gather(0)

    def octet(s1, par):
        for hs in range(2):
            c = 2 * s1 + hs
            wait_gather(hs)

            @pl.when(c + 1 < NCH)
            def _():
                wait_idx(1 - hs)
                start_gather(1 - hs)

            @pl.when(c + 2 < NCH)
            def _():
                stage_idx(c + 2, hs)

            rows = row_bufs[hs]
            for sl in range(4):
                s = s1 * 8 + hs * 4 + sl
                tb = t_bufs[sl % 2]
                if sl >= 2:
                    pltpu.make_async_copy(tb.at[:, :, pl.ds(0, 128)],
                                          out_hbm.at[0, :, wid],
                                          t_sems[sl % 2]).wait()
                else:
                    @pl.when(c >= 1)
                    def _():
                        pltpu.make_async_copy(tb.at[:, :, pl.ds(0, 128)],
                                              out_hbm.at[0, :, wid],
                                              t_sems[sl % 2]).wait()

                for g in range(4):
                    pv = pos_v[s, pl.ds(g * NL, NL)]
                    dg1, dg0 = d1v[g], d0v[g]

                    @plsc.parallel_loop(0, 128, unroll=4)
                    def _(r):
                        v = rows[sl * 128 + r, pl.ds(g * NL, NL)] + pv
                        plsc.store_scatter(tb, [dg1, dg0, lanes * 0 + r], v)

                pltpu.async_copy(tb.at[:, :, pl.ds(0, 128)],
                                 out_hbm.at[s, :, wid], t_sems[sl % 2])

    def octet_pair(k, _):
        octet(2 * k, 0)
        octet(2 * k + 1, 1)
        return 0

    lax.fori_loop(0, (NS1 - 1) // 2, octet_pair, 0)
    octet(NS1 - 1, 0)

    for i in range(2):
        pltpu.make_async_copy(t_bufs[i].at[:, :, pl.ds(0, 128)],
                              out_hbm.at[0, :, wid], t_sems[i]).wait()


@jax.jit
def kernel(inputs, token_table, position_table):
    idx4 = (inputs.T.astype(jnp.int32)
            .reshape(NS1, 8, NW, 128).transpose(0, 2, 1, 3)
            .reshape(NS1, NW, 8 * 128))
    mesh = plsc.VectorSubcoreMesh(core_axis_name="c", subcore_axis_name="s")
    out5 = pl.kernel(
        _sc_embed,
        mesh=mesh,
        out_type=jax.ShapeDtypeStruct((SEQ, 8, NW, 8, 128), jnp.float32),
        scratch_types=[
            [pltpu.VMEM((512,), jnp.int32) for _ in range(2)],
            pltpu.VMEM((SEQ, DIM), jnp.float32),
            [pltpu.VMEM((HSROWS, DIM), jnp.float32) for _ in range(2)],
            [pltpu.VMEM((8, 8, 129), jnp.float32) for _ in range(2)],
            [pltpu.SemaphoreType.DMA for _ in range(2)],
            [pltpu.SemaphoreType.DMA for _ in range(2)],
            [pltpu.SemaphoreType.DMA for _ in range(2)],
        ],
        compiler_params=pltpu.CompilerParams(use_tc_tiling_on_sc=False,
                                             needs_layout_passes=False),
    )(idx4, token_table, position_table)
    out_phys = out5.transpose(0, 1, 3, 2, 4).reshape(SEQ, DIM, BATCH)
    return out_phys.transpose(2, 0, 1)
